# Initial kernel scaffold; baseline (speedup 1.0000x reference)
import jax
import jax.numpy as jnp
from jax.experimental import pallas as pl


def kernel(x, edge_index, W1, a_src1, a_dst1, b1, W2, a_src2, a_dst2, b2):
    raise NotImplementedError("write your pallas kernel here")


# speedup vs baseline: 51.8821x; 51.8821x over previous
import functools

import jax
import jax.numpy as jnp
from jax import lax
from jax.experimental import pallas as pl
from jax.experimental.pallas import tpu as pltpu
from jax.experimental.pallas import tpu_sc as plsc

N = 10000
E = 320000
D = 128
H1 = 8
O1 = 8
F1 = H1 * O1
C = 7

NPAD = 10240
DUMMY = N
W1ROW = 80
W2ROW = 16
NTILES = 32
G = 64
ET = E + N
EPAD = ((ET + NTILES * G - 1) // (NTILES * G)) * (NTILES * G)
PER_TILE = EPAD // NTILES
CHUNKS = PER_TILE // G
ROWS_PER_TILE = NPAD // 16
NBLK = 256
GRID = NPAD // NBLK

_GATHER_DNUMS = lax.GatherDimensionNumbers(
    offset_dims=(), collapsed_slice_dims=(0,), start_index_map=(0,))


def _take16(x, idx):
    return lax.gather(x, idx[:, None], _GATHER_DNUMS, slice_sizes=(1,),
                      mode=lax.GatherScatterMode.PROMISE_IN_BOUNDS)


def _tc_layer1(x_ref, w1_ref, asrc_ref, adst_ref, s1_ref, ad1_ref):
    h = jnp.dot(x_ref[...], w1_ref[...], preferred_element_type=jnp.float32)
    a_s = jnp.dot(h, asrc_ref[...], preferred_element_type=jnp.float32)
    a_d = jnp.dot(h, adst_ref[...], preferred_element_type=jnp.float32)
    pad = jnp.zeros((NBLK, W1ROW - 8 - F1), jnp.float32)
    s1_ref[...] = jnp.concatenate([a_s, h, pad], axis=1)
    ad1_ref[...] = jnp.concatenate(
        [a_d, jnp.zeros((NBLK, 8), jnp.float32)], axis=1)


def _sc_edges1(src_r, dst_r, s1_r, ad1_r, z1_r, out_r,
               adrows, sidx, didx, srows, orows, acc_sh, sem, sem2):
    c = lax.axis_index("c")
    s = lax.axis_index("s")
    w = c * 16 + s
    base = w * PER_TILE
    row0 = s * ROWS_PER_TILE

    pltpu.sync_copy(z1_r.at[pl.ds(row0, ROWS_PER_TILE)],
                    acc_sh.at[pl.ds(row0, ROWS_PER_TILE)])
    plsc.subcore_barrier()

    iota16 = lax.iota(jnp.int32, 16)
    mask8 = iota16 < 8
    half = iota16 >> 3
    p0 = jnp.where(mask8, iota16, 0)
    p1 = half + 1
    p2 = half + 3
    p3 = half + 5
    p4 = iota16 * 0 + 7

    def chunk(g, carry):
        off = base + g * G
        pltpu.sync_copy(src_r.at[pl.ds(off, G)], sidx)
        pltpu.sync_copy(dst_r.at[pl.ds(off, G)], didx)
        d1 = pltpu.async_copy(s1_r.at[sidx], srows, sem)
        d2 = pltpu.async_copy(ad1_r.at[didx], adrows, sem2)
        d1.wait()
        d2.wait()
        for e in range(G):
                v0 = srows[e, pl.ds(0, 16)]
                v1 = srows[e, pl.ds(16, 16)]
                v2 = srows[e, pl.ds(32, 16)]
                v3 = srows[e, pl.ds(48, 16)]
                v4 = srows[e, pl.ds(64, 16)]
                ad16 = adrows[e, pl.ds(0, 16)]
                t = v0 + ad16
                ee = jnp.exp(jnp.maximum(t, 0.2 * t))
                orows[e, pl.ds(0, 16)] = jnp.where(
                    mask8, ee, _take16(ee, p0) * v0)
                orows[e, pl.ds(16, 16)] = _take16(ee, p1) * v1
                orows[e, pl.ds(32, 16)] = _take16(ee, p2) * v2
                orows[e, pl.ds(48, 16)] = _take16(ee, p3) * v3
                orows[e, pl.ds(64, 16)] = _take16(ee, p4) * v4
        pltpu.sync_copy(orows, acc_sh.at[didx], add=True)
        return carry

    lax.fori_loop(0, CHUNKS, chunk, 0)
    plsc.subcore_barrier()
    pltpu.sync_copy(acc_sh.at[pl.ds(row0, ROWS_PER_TILE)],
                    out_r.at[c, pl.ds(row0, ROWS_PER_TILE)])


def _tc_layer2_prep(accA_ref, accB_ref, b1_ref, w2_ref, r_ref,
                    asc2_ref, adc2_ref, s2_ref, ad2_ref):
    t = accA_ref[0] + accB_ref[0]
    rec8 = 1.0 / (t[:, 0:8] + 1e-16)
    num = t[:, 8:8 + F1]
    rec64 = jnp.dot(rec8, r_ref[...], preferred_element_type=jnp.float32)
    z = num * rec64 + b1_ref[...]
    z = jnp.where(z > 0, z, jnp.exp(jnp.minimum(z, 0.0)) - 1.0)
    h2 = jnp.dot(z, w2_ref[...], preferred_element_type=jnp.float32)
    a_s = jnp.dot(h2, asc2_ref[...], preferred_element_type=jnp.float32)
    a_d = jnp.dot(h2, adc2_ref[...], preferred_element_type=jnp.float32)
    s2_ref[...] = jnp.concatenate(
        [a_s, h2, jnp.zeros((NBLK, W2ROW - 1 - C), jnp.float32)], axis=1)
    ad2_ref[...] = a_d[:, 0]


def _sc_edges2(src_r, dst_r, s2_r, ad2_r, z2_r, out_r,
               ad_v, sidx, didx, srows, orows, acc_sh, sem):
    c = lax.axis_index("c")
    s = lax.axis_index("s")
    w = c * 16 + s
    base = w * PER_TILE
    row0 = s * ROWS_PER_TILE

    pltpu.sync_copy(z2_r.at[pl.ds(row0, ROWS_PER_TILE)],
                    acc_sh.at[pl.ds(row0, ROWS_PER_TILE)])
    pltpu.sync_copy(ad2_r, ad_v)
    plsc.subcore_barrier()

    iota16 = lax.iota(jnp.int32, 16)
    z16 = iota16 * 0
    lane0 = iota16 == 0

    def chunk(g, carry):
        off = base + g * G
        pltpu.sync_copy(src_r.at[pl.ds(off, G)], sidx)
        pltpu.sync_copy(dst_r.at[pl.ds(off, G)], didx)
        pltpu.async_copy(s2_r.at[sidx], srows, sem).wait()
        for j in range(G // 16):
            dvec = didx[pl.ds(j * 16, 16)]
            for k in range(16):
                e = j * 16 + k
                d_e = dvec[k]
                v0 = srows[e, pl.ds(0, 16)]
                av = ad_v[pl.ds(d_e, 16)]
                t = v0 + av[0]
                ee = jnp.exp(jnp.maximum(t, 0.2 * t))
                ee0 = _take16(ee, z16)
                orows[e, pl.ds(0, 16)] = ee0 * jnp.where(lane0, 1.0, v0)
        pltpu.sync_copy(orows, acc_sh.at[didx], add=True)
        return carry

    lax.fori_loop(0, CHUNKS, chunk, 0)
    plsc.subcore_barrier()
    pltpu.sync_copy(acc_sh.at[pl.ds(row0, ROWS_PER_TILE)],
                    out_r.at[c, pl.ds(row0, ROWS_PER_TILE)])


def _tc_final(accA_ref, accB_ref, b2_ref, o_ref):
    t = accA_ref[0] + accB_ref[0]
    den = t[:, 0:1] + 1e-16
    num = t[:, 1:1 + C]
    logits = num / den + b2_ref[...]
    m = jnp.max(logits, axis=1, keepdims=True)
    lse = jnp.log(jnp.sum(jnp.exp(logits - m), axis=1, keepdims=True))
    out = logits - m - lse
    o_ref[...] = jnp.concatenate(
        [out, jnp.zeros((NBLK, W2ROW - C), jnp.float32)], axis=1)


def kernel(x, edge_index, W1, a_src1, a_dst1, b1, W2, a_src2, a_dst2, b2):
    f32 = jnp.float32
    xp = jnp.pad(x, ((0, NPAD - N), (0, 0)))
    loop = jnp.arange(N, dtype=jnp.int32)
    padn = jnp.full((EPAD - ET,), DUMMY, jnp.int32)
    src = jnp.concatenate([edge_index[0].astype(jnp.int32), loop, padn])
    dst = jnp.concatenate([edge_index[1].astype(jnp.int32), loop, padn])
    eye8 = jnp.eye(H1, dtype=f32)
    Asrc = (a_src1[:, :, None] * eye8[:, None, :]).reshape(F1, H1)
    Adst = (a_dst1[:, :, None] * eye8[:, None, :]).reshape(F1, H1)
    R = jnp.repeat(jnp.eye(H1, dtype=f32), O1, axis=1)
    z1 = jnp.zeros((NPAD, W1ROW), f32)
    z2 = jnp.zeros((NPAD, W2ROW), f32)

    s1, ad1 = pl.pallas_call(
        _tc_layer1,
        grid=(GRID,),
        in_specs=[
            pl.BlockSpec((NBLK, D), lambda i: (i, 0)),
            pl.BlockSpec((D, F1), lambda i: (0, 0)),
            pl.BlockSpec((F1, H1), lambda i: (0, 0)),
            pl.BlockSpec((F1, H1), lambda i: (0, 0)),
        ],
        out_specs=[
            pl.BlockSpec((NBLK, W1ROW), lambda i: (i, 0)),
            pl.BlockSpec((NBLK, 16), lambda i: (i, 0)),
        ],
        out_shape=[
            jax.ShapeDtypeStruct((NPAD, W1ROW), f32),
            jax.ShapeDtypeStruct((NPAD, 16), f32),
        ],
    )(xp, W1, Asrc, Adst)

    mesh = plsc.VectorSubcoreMesh(core_axis_name="c", subcore_axis_name="s")
    acc1 = pl.kernel(
        _sc_edges1,
        out_type=jax.ShapeDtypeStruct((2, NPAD, W1ROW), f32),
        mesh=mesh,
        compiler_params=pltpu.CompilerParams(use_tc_tiling_on_sc=False),
        scratch_types=[
            pltpu.VMEM((G, 16), f32),
            pltpu.VMEM((G,), jnp.int32),
            pltpu.VMEM((G,), jnp.int32),
            pltpu.VMEM((G, W1ROW), f32),
            pltpu.VMEM((G, W1ROW), f32),
            pltpu.VMEM_SHARED((NPAD, W1ROW), f32),
            pltpu.SemaphoreType.DMA,
            pltpu.SemaphoreType.DMA,
        ],
    )(src, dst, s1, ad1, z1)

    s2, ad2 = pl.pallas_call(
        _tc_layer2_prep,
        grid=(GRID,),
        in_specs=[
            pl.BlockSpec((1, NBLK, W1ROW), lambda i: (0, i, 0)),
            pl.BlockSpec((1, NBLK, W1ROW), lambda i: (1, i, 0)),
            pl.BlockSpec((F1,), lambda i: (0,)),
            pl.BlockSpec((F1, C), lambda i: (0, 0)),
            pl.BlockSpec((H1, F1), lambda i: (0, 0)),
            pl.BlockSpec((C, 1), lambda i: (0, 0)),
            pl.BlockSpec((C, 1), lambda i: (0, 0)),
        ],
        out_specs=[
            pl.BlockSpec((NBLK, W2ROW), lambda i: (i, 0)),
            pl.BlockSpec((NBLK,), lambda i: (i,)),
        ],
        out_shape=[
            jax.ShapeDtypeStruct((NPAD, W2ROW), f32),
            jax.ShapeDtypeStruct((NPAD,), f32),
        ],
    )(acc1, acc1, b1, W2, R, a_src2.reshape(C, 1), a_dst2.reshape(C, 1))

    acc2 = pl.kernel(
        _sc_edges2,
        out_type=jax.ShapeDtypeStruct((2, NPAD, W2ROW), f32),
        mesh=mesh,
        compiler_params=pltpu.CompilerParams(use_tc_tiling_on_sc=False),
        scratch_types=[
            pltpu.VMEM((NPAD,), f32),
            pltpu.VMEM((G,), jnp.int32),
            pltpu.VMEM((G,), jnp.int32),
            pltpu.VMEM((G, W2ROW), f32),
            pltpu.VMEM((G, W2ROW), f32),
            pltpu.VMEM_SHARED((NPAD, W2ROW), f32),
            pltpu.SemaphoreType.DMA,
        ],
    )(src, dst, s2, ad2, z2)

    out = pl.pallas_call(
        _tc_final,
        grid=(GRID,),
        in_specs=[
            pl.BlockSpec((1, NBLK, W2ROW), lambda i: (0, i, 0)),
            pl.BlockSpec((1, NBLK, W2ROW), lambda i: (1, i, 0)),
            pl.BlockSpec((C,), lambda i: (0,)),
        ],
        out_specs=pl.BlockSpec((NBLK, W2ROW), lambda i: (i, 0)),
        out_shape=jax.ShapeDtypeStruct((NPAD, W2ROW), f32),
    )(acc2, acc2, b2)

    return out[:N, :C]
